# baseline (speedup 1.0000x reference)
import jax
import jax.numpy as jnp
from jax.experimental import pallas as pl
from jax.experimental.pallas import tpu as pltpu

_B = 4096
_P = 1024
_Q = 1024
_E = 8
_H = 2
_TB = _B // _H


def _moe_kernel(x_ref, w_ref, b_ref, rw_ref, out_ref, aux_ref,
                wgt_ref, imp_ref):
    h = pl.program_id(0)
    e = pl.program_id(1)

    @pl.when(e == 0)
    def _router():
        logits = jnp.dot(x_ref[...], rw_ref[...],
                         preferred_element_type=jnp.float32)
        w = jax.nn.softmax(logits, axis=-1)
        wgt_ref[...] = w
        part = jnp.sum(w, axis=0, keepdims=True)

        @pl.when(h == 0)
        def _imp0():
            imp_ref[...] = part

        @pl.when(h > 0)
        def _impn():
            imp_ref[...] = imp_ref[...] + part

        @pl.when(h == _H - 1)
        def _aux():
            imp = imp_ref[...] / jnp.float32(_B)
            aux_ref[...] = jnp.float32(_E) * jnp.sum(imp * imp, keepdims=True)

        out_ref[...] = jnp.dot(w, b_ref[...],
                               preferred_element_type=jnp.float32)

    w_all = wgt_ref[...]
    mask = jax.lax.broadcasted_iota(jnp.int32, (1, _E), 1) == e
    wcol = jnp.sum(jnp.where(mask, w_all, 0.0), axis=1, keepdims=True)

    y = jnp.dot(x_ref[...], w_ref[0], preferred_element_type=jnp.float32)
    out_ref[...] = out_ref[...] + wcol * y


def kernel(inputs, expert_w, expert_b, router_w):
    out, aux = pl.pallas_call(
        _moe_kernel,
        grid=(_H, _E),
        in_specs=[
            pl.BlockSpec((_TB, _P), lambda h, e: (h, 0)),
            pl.BlockSpec((1, _P, _Q), lambda h, e: (e, 0, 0)),
            pl.BlockSpec((_E, _Q), lambda h, e: (0, 0)),
            pl.BlockSpec((_P, _E), lambda h, e: (0, 0)),
        ],
        out_specs=[
            pl.BlockSpec((_TB, _Q), lambda h, e: (h, 0)),
            pl.BlockSpec((1, 1), lambda h, e: (0, 0)),
        ],
        out_shape=[
            jax.ShapeDtypeStruct((_B, _Q), jnp.float32),
            jax.ShapeDtypeStruct((1, 1), jnp.float32),
        ],
        scratch_shapes=[
            pltpu.VMEM((_TB, _E), jnp.float32),
            pltpu.VMEM((1, _E), jnp.float32),
        ],
        compiler_params=pltpu.CompilerParams(
            dimension_semantics=("arbitrary", "arbitrary"),
        ),
    )(inputs, expert_w, expert_b, router_w)
    return out, aux.reshape(())
